# Initial kernel scaffold; baseline (speedup 1.0000x reference)
import jax
import jax.numpy as jnp
from jax.experimental import pallas as pl


def kernel(x, edge_index, edge_weight, h, c, W_i, conv_i_W, conv_i_b, b_i, W_f, conv_f_W, conv_f_b, b_f, W_c, conv_c_W, conv_c_b, b_c, W_o, conv_o_W, conv_o_b, b_o, w_c_i, w_c_f, w_c_o):
    raise NotImplementedError("write your pallas kernel here")


# speedup vs baseline: 4.4630x; 4.4630x over previous
import functools

import jax
import jax.numpy as jnp
from jax import lax
from jax.experimental import pallas as pl
from jax.experimental.pallas import tpu as pltpu
from jax.experimental.pallas import tpu_sc as plsc

f32 = jnp.float32
i32 = jnp.int32

NC = 2
NS = 16
LANES = 16
HALF = 128
EB = 128


def _rsqrt_newton(x):
    i = plsc.bitcast(x, i32)
    y = plsc.bitcast(jnp.int32(0x5F3759DF) - (i >> 1), f32)
    for _ in range(3):
        y = y * (1.5 - 0.5 * x * y * y)
    return y


def _make_spmv(first_stage, n_pad, ew, tab_rows):
    nb = ew // EB
    rows_w = n_pad // NS
    zr = 64
    e_pad = NS * ew

    mesh = plsc.VectorSubcoreMesh(
        core_axis_name="c", subcore_axis_name="s",
        num_cores=NC, num_subcores=NS)

    out_type = [jax.ShapeDtypeStruct((NC, n_pad, HALF), f32)]
    if first_stage:
        out_type.append(jax.ShapeDtypeStruct((e_pad,), f32))

    scratch = [
        pltpu.VMEM((EB,), i32),
        pltpu.VMEM((EB,), i32),
        pltpu.VMEM((EB,), f32),
        pltpu.VMEM((EB,), f32),
        pltpu.VMEM((1, 1, EB), i32),
        pltpu.VMEM((1, 1, EB), i32),
        pltpu.VMEM((EB, HALF), f32),
        pltpu.VMEM_SHARED((n_pad, HALF), f32),
        pltpu.SemaphoreType.DMA,
    ]
    if first_stage:
        scratch += [
            pltpu.VMEM((n_pad,), f32),
            pltpu.VMEM((rows_w,), f32),
            pltpu.VMEM_SHARED((NS, n_pad), f32),
            pltpu.VMEM_SHARED((n_pad,), f32),
        ]

    def body(src_hbm, dst_hbm, wlw_hbm, tab_hbm, *rest):
        if first_stage:
            (out_hbm, lwout_hbm, srcb, dstb, wb, lwb, gidx, didx, rows,
             acc_sh, sem, nvec_v, tmp_v, deg_sh, dis_sh) = rest
        else:
            (out_hbm, srcb, dstb, wb, lwb, gidx, didx, rows,
             acc_sh, sem) = rest
            nvec_v = None

        cid = lax.axis_index("c")
        sid = lax.axis_index("s")
        ebase = sid * ew
        zeros16 = jnp.zeros((LANES,), f32)

        def _zfill(r, carry):
            for j in range(HALF // LANES):
                rows[r, pl.ds(j * LANES, LANES)] = zeros16
            return carry
        lax.fori_loop(0, EB, _zfill, 0)
        for k2 in range(rows_w // EB):
            pltpu.sync_copy(rows, acc_sh.at[pl.ds(sid * rows_w + k2 * EB, EB)])

        if first_stage:
            def _zdeg(r, carry):
                nvec_v[pl.ds(r * LANES, LANES)] = zeros16
                return carry
            lax.fori_loop(0, n_pad // LANES, _zdeg, 0)

            def _deg(b, carry):
                base = ebase + b * EB
                pltpu.sync_copy(src_hbm.at[pl.ds(base, EB)], srcb)
                pltpu.sync_copy(dst_hbm.at[pl.ds(base, EB)], dstb)
                pltpu.sync_copy(wlw_hbm.at[pl.ds(base, EB)], wb)
                for i in range(EB // LANES):
                    s = srcb[pl.ds(i * LANES, LANES)]
                    d = dstb[pl.ds(i * LANES, LANES)]
                    wv = wb[pl.ds(i * LANES, LANES)]
                    wm = jnp.where(s == d, 0.0, wv)
                    plsc.addupdate_scatter(nvec_v, [s], wm)
                return carry
            lax.fori_loop(0, nb, _deg, 0)

            pltpu.sync_copy(nvec_v, deg_sh.at[sid])
            plsc.subcore_barrier()

            def _zslice(j, carry):
                tmp_v[pl.ds(j * LANES, LANES)] = zeros16
                return carry
            lax.fori_loop(0, rows_w // LANES, _zslice, 0)
            for t in range(NS):
                pltpu.sync_copy(deg_sh.at[t, pl.ds(sid * rows_w, rows_w)],
                                nvec_v.at[pl.ds(0, rows_w)])

                def _acc(j, carry):
                    sl = pl.ds(j * LANES, LANES)
                    tmp_v[sl] = tmp_v[sl] + nvec_v[sl]
                    return carry
                lax.fori_loop(0, rows_w // LANES, _acc, 0)

            def _dis(j, carry):
                sl = pl.ds(j * LANES, LANES)
                acc = tmp_v[sl]
                y = _rsqrt_newton(jnp.maximum(acc, 1e-12))
                y = jnp.where(acc > 0.0, y, 0.0)
                tmp_v[sl] = y
                return carry
            lax.fori_loop(0, rows_w // LANES, _dis, 0)

            pltpu.sync_copy(tmp_v, dis_sh.at[pl.ds(sid * rows_w, rows_w)])
            plsc.subcore_barrier()
            pltpu.sync_copy(dis_sh, nvec_v)

        plsc.subcore_barrier()

        if first_stage:
            gmul, goff = 2, 1
        else:
            gmul, goff = 1, n_pad

        def _mainb(b, carry):
            base = ebase + b * EB
            pltpu.sync_copy(src_hbm.at[pl.ds(base, EB)], srcb)
            pltpu.sync_copy(dst_hbm.at[pl.ds(base, EB)], dstb)
            pltpu.sync_copy(wlw_hbm.at[pl.ds(base, EB)], wb)
            for i in range(EB // LANES):
                s = srcb[pl.ds(i * LANES, LANES)]
                d = dstb[pl.ds(i * LANES, LANES)]
                if first_stage:
                    wv = wb[pl.ds(i * LANES, LANES)]
                    wm = jnp.where(s == d, 0.0, wv)
                    lw = -(plsc.load_gather(nvec_v, [s]) * wm
                           * plsc.load_gather(nvec_v, [d]))
                    lwb[pl.ds(i * LANES, LANES)] = lw
                gidx[0, 0, pl.ds(i * LANES, LANES)] = gmul * s + cid * goff
                didx[0, 0, pl.ds(i * LANES, LANES)] = d
            if first_stage:
                @pl.when(cid == 0)
                def _store_lw():
                    pltpu.sync_copy(lwb, lwout_hbm.at[pl.ds(base, EB)])
            pltpu.async_copy(tab_hbm.at[gidx.at[0, 0]], rows, sem).wait()

            lwsrc = lwb if first_stage else wb

            def _scale(g, carry2):
                lwvec = lwsrc[pl.ds(g * LANES, LANES)]
                for k in range(LANES):
                    fct = lwvec[k]
                    e2 = g * LANES + k
                    for j in range(HALF // LANES):
                        sl = pl.ds(j * LANES, LANES)
                        rows[e2, sl] = rows[e2, sl] * fct
                return carry2
            lax.fori_loop(0, EB // LANES, _scale, 0)

            pltpu.async_copy(rows, acc_sh.at[didx.at[0, 0]], sem,
                             add=True).wait()
            return carry
        lax.fori_loop(0, nb, _mainb, 0)

        plsc.subcore_barrier()
        pltpu.sync_copy(acc_sh.at[pl.ds(sid * rows_w, rows_w)],
                        out_hbm.at[cid, pl.ds(sid * rows_w, rows_w)])

    return pl.kernel(body, out_type=out_type, mesh=mesh,
                     scratch_types=scratch,
                     compiler_params=pltpu.CompilerParams(
                         needs_layout_passes=False))


def _dense_gates(x, h, c, t1, tx, wx, wh, w1a, w1b, w2a, w2b, bias,
                 wci, wcf, wco):
    n, fin = x.shape
    hd = c.shape[1]
    m = 1000
    assert n % m == 0

    def body(x_ref, h_ref, c_ref, t1_ref, tx_ref, wx_ref, wh_ref, w1a_ref,
             w1b_ref, w2a_ref, w2b_ref, b_ref, wci_ref, wcf_ref, wco_ref,
             hn_ref, cn_ref):
        z = jnp.dot(x_ref[...], wx_ref[...], preferred_element_type=f32)
        z = z + jnp.dot(h_ref[...], wh_ref[...], preferred_element_type=f32)
        z = z + jnp.dot(t1_ref[0], w1a_ref[...], preferred_element_type=f32)
        z = z + jnp.dot(t1_ref[1], w1b_ref[...], preferred_element_type=f32)
        z = z + jnp.dot(tx_ref[0], w2a_ref[...], preferred_element_type=f32)
        z = z + jnp.dot(tx_ref[1], w2b_ref[...], preferred_element_type=f32)
        z = z + b_ref[...]
        cc = c_ref[...]
        ig = jax.nn.sigmoid(z[:, 0:hd] + wci_ref[...] * cc)
        fg = jax.nn.sigmoid(z[:, hd:2 * hd] + wcf_ref[...] * cc)
        tg = jnp.tanh(z[:, 2 * hd:3 * hd])
        cn = fg * cc + ig * tg
        og = jax.nn.sigmoid(z[:, 3 * hd:4 * hd] + wco_ref[...] * cn)
        hn_ref[...] = og * jnp.tanh(cn)
        cn_ref[...] = cn

    g4 = 4 * hd
    const = lambda shape: pl.BlockSpec(shape, lambda i: tuple(0 for _ in shape))
    return pl.pallas_call(
        body,
        grid=(n // m,),
        in_specs=[
            pl.BlockSpec((m, fin), lambda i: (i, 0)),
            pl.BlockSpec((m, hd), lambda i: (i, 0)),
            pl.BlockSpec((m, hd), lambda i: (i, 0)),
            pl.BlockSpec((NC, m, HALF), lambda i: (0, i, 0)),
            pl.BlockSpec((NC, m, HALF), lambda i: (0, i, 0)),
            const((fin, g4)),
            const((hd, g4)),
            const((HALF, g4)),
            const((HALF, g4)),
            const((HALF, g4)),
            const((HALF, g4)),
            const((1, g4)),
            const((1, hd)),
            const((1, hd)),
            const((1, hd)),
        ],
        out_specs=[pl.BlockSpec((m, hd), lambda i: (i, 0)),
                   pl.BlockSpec((m, hd), lambda i: (i, 0))],
        out_shape=[jax.ShapeDtypeStruct((n, hd), f32),
                   jax.ShapeDtypeStruct((n, hd), f32)],
    )(x, h, c, t1, tx, wx, wh, w1a, w1b, w2a, w2b, bias, wci, wcf, wco)


def kernel(x, edge_index, edge_weight, h, c, W_i, conv_i_W, conv_i_b, b_i,
           W_f, conv_f_W, conv_f_b, b_f, W_c, conv_c_W, conv_c_b, b_c,
           W_o, conv_o_W, conv_o_b, b_o, w_c_i, w_c_f, w_c_o):
    n = x.shape[0]
    e = edge_index.shape[1]
    hd = h.shape[1]
    assert hd == 2 * HALF

    n_pad = -(-n // (NS * 64)) * (NS * 64)
    ew = -(-e // (NS * EB)) * EB
    e_pad = NS * ew

    src = jnp.pad(edge_index[0], (0, e_pad - e))
    dst = jnp.pad(edge_index[1], (0, e_pad - e))
    w = jnp.pad(edge_weight, (0, e_pad - e))

    spmv1 = _make_spmv(True, n_pad, ew, 2 * n)
    spmv2 = _make_spmv(False, n_pad, ew, 2 * n_pad)

    t1p, lw = spmv1(src, dst, w, h.reshape(2 * n, HALF))
    [txp] = spmv2(src, dst, lw, t1p.reshape(2 * n_pad, HALF))
    t1 = t1p[:, :n, :]
    tx = txp[:, :n, :]

    wx = jnp.concatenate([W_i, W_f, W_c, W_o], axis=1)
    wh = jnp.concatenate([conv_i_W[0] - conv_i_W[2], conv_f_W[0] - conv_f_W[2],
                          conv_c_W[0] - conv_c_W[2], conv_o_W[0] - conv_o_W[2]],
                         axis=1)
    w1 = jnp.concatenate([conv_i_W[1], conv_f_W[1], conv_c_W[1], conv_o_W[1]],
                         axis=1)
    w2 = jnp.concatenate([2.0 * conv_i_W[2], 2.0 * conv_f_W[2],
                          2.0 * conv_c_W[2], 2.0 * conv_o_W[2]], axis=1)
    bias = jnp.concatenate([conv_i_b + b_i, conv_f_b + b_f, conv_c_b + b_c,
                            conv_o_b + b_o])[None, :]

    hn, cn = _dense_gates(x, h, c, t1, tx, wx, wh, w1[:HALF], w1[HALF:],
                          w2[:HALF], w2[HALF:], bias, w_c_i[None, :],
                          w_c_f[None, :], w_c_o[None, :])
    return hn, cn


# speedup vs baseline: 4.6112x; 1.0332x over previous
import functools

import jax
import jax.numpy as jnp
from jax import lax
from jax.experimental import pallas as pl
from jax.experimental.pallas import tpu as pltpu
from jax.experimental.pallas import tpu_sc as plsc

f32 = jnp.float32
i32 = jnp.int32

NC = 2
NS = 16
LANES = 16
HALF = 128
EB = 128


def _rsqrt_newton(x):
    i = plsc.bitcast(x, i32)
    y = plsc.bitcast(jnp.int32(0x5F3759DF) - (i >> 1), f32)
    for _ in range(3):
        y = y * (1.5 - 0.5 * x * y * y)
    return y


def _make_spmv(first_stage, n_pad, ew, tab_rows):
    nb = ew // EB
    assert nb % 2 == 0
    rows_w = n_pad // NS
    e_pad = NS * ew

    mesh = plsc.VectorSubcoreMesh(
        core_axis_name="c", subcore_axis_name="s",
        num_cores=NC, num_subcores=NS)

    out_type = [jax.ShapeDtypeStruct((NC, n_pad, HALF), f32)]
    if first_stage:
        out_type.append(jax.ShapeDtypeStruct((e_pad,), f32))

    scratch = [
        pltpu.VMEM((EB,), i32),
        pltpu.VMEM((EB,), i32),
        pltpu.VMEM((EB,), f32),
        pltpu.VMEM((2, EB), f32),
        pltpu.VMEM((2, 1, EB), i32),
        pltpu.VMEM((2, 1, EB), i32),
        pltpu.VMEM((2, EB, HALF), f32),
        pltpu.VMEM_SHARED((n_pad, HALF), f32),
        pltpu.SemaphoreType.DMA,
        pltpu.SemaphoreType.DMA,
        pltpu.SemaphoreType.DMA,
        pltpu.SemaphoreType.DMA,
    ]
    if first_stage:
        scratch += [
            pltpu.VMEM((n_pad,), f32),
            pltpu.VMEM((rows_w,), f32),
            pltpu.HBM((NC, NS, n_pad), f32),
            pltpu.VMEM_SHARED((n_pad,), f32),
        ]

    def body(src_hbm, dst_hbm, wlw_hbm, tab_hbm, *rest):
        if first_stage:
            (out_hbm, lwout_hbm, srcb, dstb, wb, lwb, gidx, didx, rows,
             acc_sh, sem_g0, sem_g1, sem_s0, sem_s1,
             nvec_v, tmp_v, deg_st, dis_sh) = rest
        else:
            (out_hbm, srcb, dstb, wb, lwb, gidx, didx, rows,
             acc_sh, sem_g0, sem_g1, sem_s0, sem_s1) = rest
            nvec_v = None

        cid = lax.axis_index("c")
        sid = lax.axis_index("s")
        ebase = sid * ew
        zeros16 = jnp.zeros((LANES,), f32)

        def _zfill(r, carry):
            for j in range(HALF // LANES):
                rows[0, r, pl.ds(j * LANES, LANES)] = zeros16
            return carry
        lax.fori_loop(0, EB, _zfill, 0)
        for k2 in range(rows_w // EB):
            pltpu.sync_copy(rows.at[0],
                            acc_sh.at[pl.ds(sid * rows_w + k2 * EB, EB)])

        if first_stage:
            def _zdeg(r, carry):
                nvec_v[pl.ds(r * LANES, LANES)] = zeros16
                return carry
            lax.fori_loop(0, n_pad // LANES, _zdeg, 0)

            def _deg(b, carry):
                base = ebase + b * EB
                pltpu.sync_copy(src_hbm.at[pl.ds(base, EB)], srcb)
                pltpu.sync_copy(dst_hbm.at[pl.ds(base, EB)], dstb)
                pltpu.sync_copy(wlw_hbm.at[pl.ds(base, EB)], wb)
                for i in range(EB // LANES):
                    s = srcb[pl.ds(i * LANES, LANES)]
                    d = dstb[pl.ds(i * LANES, LANES)]
                    wv = wb[pl.ds(i * LANES, LANES)]
                    wm = jnp.where(s == d, 0.0, wv)
                    plsc.addupdate_scatter(nvec_v, [s], wm)
                return carry
            lax.fori_loop(0, nb, _deg, 0)

            pltpu.sync_copy(nvec_v, deg_st.at[cid, sid])
            plsc.subcore_barrier()

            def _zslice(j, carry):
                tmp_v[pl.ds(j * LANES, LANES)] = zeros16
                return carry
            lax.fori_loop(0, rows_w // LANES, _zslice, 0)
            for t in range(NS):
                pltpu.sync_copy(deg_st.at[cid, t, pl.ds(sid * rows_w, rows_w)],
                                nvec_v.at[pl.ds(0, rows_w)])

                def _acc(j, carry):
                    sl = pl.ds(j * LANES, LANES)
                    tmp_v[sl] = tmp_v[sl] + nvec_v[sl]
                    return carry
                lax.fori_loop(0, rows_w // LANES, _acc, 0)

            def _dis(j, carry):
                sl = pl.ds(j * LANES, LANES)
                acc = tmp_v[sl]
                y = _rsqrt_newton(jnp.maximum(acc, 1e-12))
                y = jnp.where(acc > 0.0, y, 0.0)
                tmp_v[sl] = y
                return carry
            lax.fori_loop(0, rows_w // LANES, _dis, 0)

            pltpu.sync_copy(tmp_v, dis_sh.at[pl.ds(sid * rows_w, rows_w)])
            plsc.subcore_barrier()
            pltpu.sync_copy(dis_sh, nvec_v)

        plsc.subcore_barrier()

        if first_stage:
            gmul, goff = 2, 1
        else:
            gmul, goff = 1, n_pad

        def _build(b, slot):
            base = ebase + b * EB
            pltpu.sync_copy(src_hbm.at[pl.ds(base, EB)], srcb)
            pltpu.sync_copy(dst_hbm.at[pl.ds(base, EB)], dstb)
            pltpu.sync_copy(wlw_hbm.at[pl.ds(base, EB)], wb)
            for i in range(EB // LANES):
                s = srcb[pl.ds(i * LANES, LANES)]
                d = dstb[pl.ds(i * LANES, LANES)]
                if first_stage:
                    wv = wb[pl.ds(i * LANES, LANES)]
                    wm = jnp.where(s == d, 0.0, wv)
                    lw = -(plsc.load_gather(nvec_v, [s]) * wm
                           * plsc.load_gather(nvec_v, [d]))
                    lwb[slot, pl.ds(i * LANES, LANES)] = lw
                else:
                    lwb[slot, pl.ds(i * LANES, LANES)] = \
                        wb[pl.ds(i * LANES, LANES)]
                gidx[slot, 0, pl.ds(i * LANES, LANES)] = gmul * s + cid * goff
                didx[slot, 0, pl.ds(i * LANES, LANES)] = d
            if first_stage:
                @pl.when(cid == 0)
                def _store_lw():
                    pltpu.sync_copy(lwb.at[slot], lwout_hbm.at[pl.ds(base, EB)])

        def _scale(slot):
            def _sc16(g, carry2):
                lwvec = lwb[slot, pl.ds(g * LANES, LANES)]
                for k in range(LANES):
                    fct = lwvec[k]
                    e2 = g * LANES + k
                    for j in range(HALF // LANES):
                        sl = pl.ds(j * LANES, LANES)
                        rows[slot, e2, sl] = rows[slot, e2, sl] * fct
                return carry2
            lax.fori_loop(0, EB // LANES, _sc16, 0)

        sem_g = (sem_g0, sem_g1)
        sem_s = (sem_s0, sem_s1)

        def _gather_start(slot):
            pltpu.async_copy(tab_hbm.at[gidx.at[slot, 0]], rows.at[slot],
                             sem_g[slot])

        def _gather_wait(slot):
            pltpu.make_async_copy(tab_hbm.at[gidx.at[slot, 0]], rows.at[slot],
                                  sem_g[slot]).wait()

        def _scatter_start(slot):
            pltpu.async_copy(rows.at[slot], acc_sh.at[didx.at[slot, 0]],
                             sem_s[slot], add=True)

        def _scatter_wait(slot):
            pltpu.make_async_copy(rows.at[slot], acc_sh.at[didx.at[slot, 0]],
                                  sem_s[slot]).wait()

        def _pair(bb, carry):
            for slot in (0, 1):
                @pl.when(bb > 0)
                def _drain():
                    _scatter_wait(slot)
                _build(bb + slot, slot)
                _gather_start(slot)
            for slot in (0, 1):
                _gather_wait(slot)
                _scale(slot)
                _scatter_start(slot)
            return carry
        lax.fori_loop(0, nb // 2, lambda p, c: _pair(p * 2, c), 0)
        _scatter_wait(0)
        _scatter_wait(1)

        plsc.subcore_barrier()
        pltpu.sync_copy(acc_sh.at[pl.ds(sid * rows_w, rows_w)],
                        out_hbm.at[cid, pl.ds(sid * rows_w, rows_w)])

    return pl.kernel(body, out_type=out_type, mesh=mesh,
                     scratch_types=scratch,
                     compiler_params=pltpu.CompilerParams(
                         needs_layout_passes=False))


def _dense_gates(x, h, c, t1, tx, wx, wh, w1a, w1b, w2a, w2b, bias,
                 wci, wcf, wco):
    n, fin = x.shape
    hd = c.shape[1]
    m = 1000
    assert n % m == 0

    def body(x_ref, h_ref, c_ref, t1_ref, tx_ref, wx_ref, wh_ref, w1a_ref,
             w1b_ref, w2a_ref, w2b_ref, b_ref, wci_ref, wcf_ref, wco_ref,
             hn_ref, cn_ref):
        z = jnp.dot(x_ref[...], wx_ref[...], preferred_element_type=f32)
        z = z + jnp.dot(h_ref[...], wh_ref[...], preferred_element_type=f32)
        z = z + jnp.dot(t1_ref[0], w1a_ref[...], preferred_element_type=f32)
        z = z + jnp.dot(t1_ref[1], w1b_ref[...], preferred_element_type=f32)
        z = z + jnp.dot(tx_ref[0], w2a_ref[...], preferred_element_type=f32)
        z = z + jnp.dot(tx_ref[1], w2b_ref[...], preferred_element_type=f32)
        z = z + b_ref[...]
        cc = c_ref[...]
        ig = jax.nn.sigmoid(z[:, 0:hd] + wci_ref[...] * cc)
        fg = jax.nn.sigmoid(z[:, hd:2 * hd] + wcf_ref[...] * cc)
        tg = jnp.tanh(z[:, 2 * hd:3 * hd])
        cn = fg * cc + ig * tg
        og = jax.nn.sigmoid(z[:, 3 * hd:4 * hd] + wco_ref[...] * cn)
        hn_ref[...] = og * jnp.tanh(cn)
        cn_ref[...] = cn

    g4 = 4 * hd
    const = lambda shape: pl.BlockSpec(shape, lambda i: tuple(0 for _ in shape))
    return pl.pallas_call(
        body,
        grid=(n // m,),
        in_specs=[
            pl.BlockSpec((m, fin), lambda i: (i, 0)),
            pl.BlockSpec((m, hd), lambda i: (i, 0)),
            pl.BlockSpec((m, hd), lambda i: (i, 0)),
            pl.BlockSpec((NC, m, HALF), lambda i: (0, i, 0)),
            pl.BlockSpec((NC, m, HALF), lambda i: (0, i, 0)),
            const((fin, g4)),
            const((hd, g4)),
            const((HALF, g4)),
            const((HALF, g4)),
            const((HALF, g4)),
            const((HALF, g4)),
            const((1, g4)),
            const((1, hd)),
            const((1, hd)),
            const((1, hd)),
        ],
        out_specs=[pl.BlockSpec((m, hd), lambda i: (i, 0)),
                   pl.BlockSpec((m, hd), lambda i: (i, 0))],
        out_shape=[jax.ShapeDtypeStruct((n, hd), f32),
                   jax.ShapeDtypeStruct((n, hd), f32)],
    )(x, h, c, t1, tx, wx, wh, w1a, w1b, w2a, w2b, bias, wci, wcf, wco)


def kernel(x, edge_index, edge_weight, h, c, W_i, conv_i_W, conv_i_b, b_i,
           W_f, conv_f_W, conv_f_b, b_f, W_c, conv_c_W, conv_c_b, b_c,
           W_o, conv_o_W, conv_o_b, b_o, w_c_i, w_c_f, w_c_o):
    n = x.shape[0]
    e = edge_index.shape[1]
    hd = h.shape[1]
    assert hd == 2 * HALF

    n_pad = -(-n // (NS * EB)) * (NS * EB)
    ew = -(-e // (NS * 2 * EB)) * (2 * EB)
    e_pad = NS * ew

    src = jnp.pad(edge_index[0], (0, e_pad - e))
    dst = jnp.pad(edge_index[1], (0, e_pad - e))
    w = jnp.pad(edge_weight, (0, e_pad - e))

    spmv1 = _make_spmv(True, n_pad, ew, 2 * n)
    spmv2 = _make_spmv(False, n_pad, ew, 2 * n_pad)

    t1p, lw = spmv1(src, dst, w, h.reshape(2 * n, HALF))
    [txp] = spmv2(src, dst, lw, t1p.reshape(2 * n_pad, HALF))
    t1 = t1p[:, :n, :]
    tx = txp[:, :n, :]

    wx = jnp.concatenate([W_i, W_f, W_c, W_o], axis=1)
    wh = jnp.concatenate([conv_i_W[0] - conv_i_W[2], conv_f_W[0] - conv_f_W[2],
                          conv_c_W[0] - conv_c_W[2], conv_o_W[0] - conv_o_W[2]],
                         axis=1)
    w1 = jnp.concatenate([conv_i_W[1], conv_f_W[1], conv_c_W[1], conv_o_W[1]],
                         axis=1)
    w2 = jnp.concatenate([2.0 * conv_i_W[2], 2.0 * conv_f_W[2],
                          2.0 * conv_c_W[2], 2.0 * conv_o_W[2]], axis=1)
    bias = jnp.concatenate([conv_i_b + b_i, conv_f_b + b_f, conv_c_b + b_c,
                            conv_o_b + b_o])[None, :]

    hn, cn = _dense_gates(x, h, c, t1, tx, wx, wh, w1[:HALF], w1[HALF:],
                          w2[:HALF], w2[HALF:], bias, w_c_i[None, :],
                          w_c_f[None, :], w_c_o[None, :])
    return hn, cn


# speedup vs baseline: 5.3135x; 1.1523x over previous
import jax
import jax.numpy as jnp
from jax import lax
from jax.experimental import pallas as pl
from jax.experimental.pallas import tpu as pltpu
from jax.experimental.pallas import tpu_sc as plsc

f32 = jnp.float32
i32 = jnp.int32

NC = 2
NS = 16
LANES = 16
HALF = 128
EB = 128


def _rsqrt_newton(x):
    i = plsc.bitcast(x, i32)
    y = plsc.bitcast(jnp.int32(0x5F3759DF) - (i >> 1), f32)
    for _ in range(3):
        y = y * (1.5 - 0.5 * x * y * y)
    return y


def _make_spmv(first_stage, n_pad, ew):
    nb = ew // EB
    assert nb % 2 == 0
    rows_w = n_pad // NS
    mesh = plsc.VectorSubcoreMesh(
        core_axis_name="c", subcore_axis_name="s",
        num_cores=NC, num_subcores=NS)

    out_type = [jax.ShapeDtypeStruct((NC, n_pad, HALF), f32)]
    if first_stage:
        out_type.append(jax.ShapeDtypeStruct((n_pad,), f32))

    scratch = [
        pltpu.VMEM((2, 3 * EB), i32),
        pltpu.VMEM((2, EB), f32),
        pltpu.VMEM((2, 1, EB), i32),
        pltpu.VMEM((2, 1, EB), i32),
        pltpu.VMEM((2, EB, HALF), f32),
        pltpu.VMEM((n_pad,), f32),
        pltpu.VMEM_SHARED((n_pad, HALF), f32),
        pltpu.SemaphoreType.DMA,
        pltpu.SemaphoreType.DMA,
        pltpu.SemaphoreType.DMA,
        pltpu.SemaphoreType.DMA,
    ]
    if first_stage:
        scratch += [
            pltpu.VMEM((rows_w,), f32),
            pltpu.HBM((NC, NS, n_pad), f32),
            pltpu.VMEM_SHARED((n_pad,), f32),
        ]

    def body(pk_hbm, *rest):
        if first_stage:
            (tab, out_hbm, dis_hbm, pkb, lwb, gidx, didx, rows, dis_v,
             acc_sh, sem_g0, sem_g1, sem_s0, sem_s1,
             tmp_v, deg_st, dis_sh) = rest
            dis_in = None
        else:
            (dis_in, tab, out_hbm, pkb, lwb, gidx, didx, rows, dis_v,
             acc_sh, sem_g0, sem_g1, sem_s0, sem_s1) = rest

        cid = lax.axis_index("c")
        sid = lax.axis_index("s")
        zeros16 = jnp.zeros((LANES,), f32)

        def _zfill(r, carry):
            for j in range(HALF // LANES):
                rows[0, r, pl.ds(j * LANES, LANES)] = zeros16
            return carry
        lax.fori_loop(0, EB, _zfill, 0)
        for k2 in range(rows_w // EB):
            pltpu.sync_copy(rows.at[0],
                            acc_sh.at[pl.ds(sid * rows_w + k2 * EB, EB)])

        if first_stage:
            def _zdeg(r, carry):
                dis_v[pl.ds(r * LANES, LANES)] = zeros16
                return carry
            lax.fori_loop(0, n_pad // LANES, _zdeg, 0)

            def _deg(p, carry):
                pltpu.sync_copy(pk_hbm.at[sid, pl.ds(p * 2, 2)], pkb)
                for slot in range(2):
                    for i in range(EB // LANES):
                        off = i * LANES
                        s = pkb[slot, pl.ds(off, LANES)]
                        d = pkb[slot, pl.ds(EB + off, LANES)]
                        wv = plsc.bitcast(pkb[slot, pl.ds(2 * EB + off, LANES)],
                                          f32)
                        wm = jnp.where(s == d, 0.0, wv)
                        plsc.addupdate_scatter(dis_v, [s], wm)
                return carry
            lax.fori_loop(0, nb // 2, _deg, 0)

            pltpu.sync_copy(dis_v, deg_st.at[cid, sid])
            plsc.subcore_barrier()

            def _zslice(j, carry):
                tmp_v[pl.ds(j * LANES, LANES)] = zeros16
                return carry
            lax.fori_loop(0, rows_w // LANES, _zslice, 0)
            for t in range(NS):
                pltpu.sync_copy(deg_st.at[cid, t, pl.ds(sid * rows_w, rows_w)],
                                dis_v.at[pl.ds(0, rows_w)])

                def _acc(j, carry):
                    sl = pl.ds(j * LANES, LANES)
                    tmp_v[sl] = tmp_v[sl] + dis_v[sl]
                    return carry
                lax.fori_loop(0, rows_w // LANES, _acc, 0)

            def _dis(j, carry):
                sl = pl.ds(j * LANES, LANES)
                acc = tmp_v[sl]
                y = _rsqrt_newton(jnp.maximum(acc, 1e-12))
                y = jnp.where(acc > 0.0, y, 0.0)
                tmp_v[sl] = y
                return carry
            lax.fori_loop(0, rows_w // LANES, _dis, 0)

            pltpu.sync_copy(tmp_v, dis_sh.at[pl.ds(sid * rows_w, rows_w)])
            plsc.subcore_barrier()
            pltpu.sync_copy(dis_sh, dis_v)

            @pl.when((cid == 0) & (sid == 0))
            def _export_dis():
                pltpu.sync_copy(dis_sh, dis_hbm)
        else:
            pltpu.sync_copy(dis_in, dis_v)

        plsc.subcore_barrier()

        if first_stage:
            gmul, goff = 2, 1
        else:
            gmul, goff = 1, n_pad

        def _build(slot):
            for i in range(EB // LANES):
                off = i * LANES
                sl = pl.ds(off, LANES)
                s = pkb[slot, sl]
                d = pkb[slot, pl.ds(EB + off, LANES)]
                wv = plsc.bitcast(pkb[slot, pl.ds(2 * EB + off, LANES)], f32)
                wm = jnp.where(s == d, 0.0, wv)
                lwb[slot, sl] = -(plsc.load_gather(dis_v, [s]) * wm
                                  * plsc.load_gather(dis_v, [d]))
                gidx[slot, 0, sl] = gmul * s + cid * goff
                didx[slot, 0, sl] = d

        def _scale(slot):
            def _sc16(g, carry2):
                lwvec = lwb[slot, pl.ds(g * LANES, LANES)]
                for k in range(LANES):
                    fct = lwvec[k]
                    e2 = g * LANES + k
                    for j in range(HALF // LANES):
                        sl = pl.ds(j * LANES, LANES)
                        rows[slot, e2, sl] = rows[slot, e2, sl] * fct
                return carry2
            lax.fori_loop(0, EB // LANES, _sc16, 0)

        sem_g = (sem_g0, sem_g1)
        sem_s = (sem_s0, sem_s1)

        def _gather_start(slot):
            pltpu.async_copy(tab.at[gidx.at[slot, 0]], rows.at[slot],
                             sem_g[slot])

        def _gather_wait(slot):
            pltpu.make_async_copy(tab.at[gidx.at[slot, 0]], rows.at[slot],
                                  sem_g[slot]).wait()

        def _scatter_start(slot):
            pltpu.async_copy(rows.at[slot], acc_sh.at[didx.at[slot, 0]],
                             sem_s[slot], add=True)

        def _scatter_wait(slot):
            pltpu.make_async_copy(rows.at[slot], acc_sh.at[didx.at[slot, 0]],
                                  sem_s[slot]).wait()

        def _pair(p, carry):
            @pl.when(p > 0)
            def _drain():
                _scatter_wait(0)
                _scatter_wait(1)
            pltpu.sync_copy(pk_hbm.at[sid, pl.ds(p * 2, 2)], pkb)
            for slot in (0, 1):
                _build(slot)
                _gather_start(slot)
            for slot in (0, 1):
                _gather_wait(slot)
                _scale(slot)
                _scatter_start(slot)
            return carry
        lax.fori_loop(0, nb // 2, _pair, 0)
        _scatter_wait(0)
        _scatter_wait(1)

        plsc.subcore_barrier()
        pltpu.sync_copy(acc_sh.at[pl.ds(sid * rows_w, rows_w)],
                        out_hbm.at[cid, pl.ds(sid * rows_w, rows_w)])

    return pl.kernel(body, out_type=out_type, mesh=mesh,
                     scratch_types=scratch,
                     compiler_params=pltpu.CompilerParams(
                         needs_layout_passes=False))


def _dense_gates(x, h, c, t1, tx, wx, wh, w1a, w1b, w2a, w2b, bias,
                 wci, wcf, wco):
    n, fin = x.shape
    hd = c.shape[1]
    m = 1000
    assert n % m == 0
    n_pad = t1.shape[1]

    def body(x_ref, h_ref, c_ref, t1_ref, tx_ref, wx_ref, wh_ref, w1a_ref,
             w1b_ref, w2a_ref, w2b_ref, b_ref, wci_ref, wcf_ref, wco_ref,
             hn_ref, cn_ref):
        z = jnp.dot(x_ref[...], wx_ref[...], preferred_element_type=f32)
        z = z + jnp.dot(h_ref[...], wh_ref[...], preferred_element_type=f32)
        z = z + jnp.dot(t1_ref[0], w1a_ref[...], preferred_element_type=f32)
        z = z + jnp.dot(t1_ref[1], w1b_ref[...], preferred_element_type=f32)
        z = z + jnp.dot(tx_ref[0], w2a_ref[...], preferred_element_type=f32)
        z = z + jnp.dot(tx_ref[1], w2b_ref[...], preferred_element_type=f32)
        z = z + b_ref[...]
        cc = c_ref[...]
        ig = jax.nn.sigmoid(z[:, 0:hd] + wci_ref[...] * cc)
        fg = jax.nn.sigmoid(z[:, hd:2 * hd] + wcf_ref[...] * cc)
        tg = jnp.tanh(z[:, 2 * hd:3 * hd])
        cn = fg * cc + ig * tg
        og = jax.nn.sigmoid(z[:, 3 * hd:4 * hd] + wco_ref[...] * cn)
        hn_ref[...] = og * jnp.tanh(cn)
        cn_ref[...] = cn

    g4 = 4 * hd
    const = lambda shape: pl.BlockSpec(shape, lambda i: tuple(0 for _ in shape))
    return pl.pallas_call(
        body,
        grid=(n // m,),
        in_specs=[
            pl.BlockSpec((m, fin), lambda i: (i, 0)),
            pl.BlockSpec((m, hd), lambda i: (i, 0)),
            pl.BlockSpec((m, hd), lambda i: (i, 0)),
            pl.BlockSpec((NC, m, HALF), lambda i: (0, i, 0)),
            pl.BlockSpec((NC, m, HALF), lambda i: (0, i, 0)),
            const((fin, g4)),
            const((hd, g4)),
            const((HALF, g4)),
            const((HALF, g4)),
            const((HALF, g4)),
            const((HALF, g4)),
            const((1, g4)),
            const((1, hd)),
            const((1, hd)),
            const((1, hd)),
        ],
        out_specs=[pl.BlockSpec((m, hd), lambda i: (i, 0)),
                   pl.BlockSpec((m, hd), lambda i: (i, 0))],
        out_shape=[jax.ShapeDtypeStruct((n, hd), f32),
                   jax.ShapeDtypeStruct((n, hd), f32)],
    )(x, h, c, t1, tx, wx, wh, w1a, w1b, w2a, w2b, bias, wci, wcf, wco)


def kernel(x, edge_index, edge_weight, h, c, W_i, conv_i_W, conv_i_b, b_i,
           W_f, conv_f_W, conv_f_b, b_f, W_c, conv_c_W, conv_c_b, b_c,
           W_o, conv_o_W, conv_o_b, b_o, w_c_i, w_c_f, w_c_o):
    n = x.shape[0]
    e = edge_index.shape[1]
    hd = h.shape[1]
    assert hd == 2 * HALF

    n_pad = -(-n // (NS * EB)) * (NS * EB)
    ew = -(-e // (NS * 2 * EB)) * (2 * EB)
    e_pad = NS * ew
    nb = ew // EB

    src = jnp.pad(edge_index[0], (0, e_pad - e))
    dst = jnp.pad(edge_index[1], (0, e_pad - e))
    w = jnp.pad(edge_weight, (0, e_pad - e))

    pk = jnp.stack([src, dst, lax.bitcast_convert_type(w, i32)])
    pk = pk.reshape(3, NS, nb, EB).transpose(1, 2, 0, 3).reshape(NS, nb, 3 * EB)

    spmv1 = _make_spmv(True, n_pad, ew)
    spmv2 = _make_spmv(False, n_pad, ew)

    t1p, dis = spmv1(pk, h.reshape(2 * n, HALF))
    [txp] = spmv2(pk, dis, t1p.reshape(2 * n_pad, HALF))

    wx = jnp.concatenate([W_i, W_f, W_c, W_o], axis=1)
    wh = jnp.concatenate([conv_i_W[0] - conv_i_W[2], conv_f_W[0] - conv_f_W[2],
                          conv_c_W[0] - conv_c_W[2], conv_o_W[0] - conv_o_W[2]],
                         axis=1)
    w1 = jnp.concatenate([conv_i_W[1], conv_f_W[1], conv_c_W[1], conv_o_W[1]],
                         axis=1)
    w2 = jnp.concatenate([2.0 * conv_i_W[2], 2.0 * conv_f_W[2],
                          2.0 * conv_c_W[2], 2.0 * conv_o_W[2]], axis=1)
    bias = jnp.concatenate([conv_i_b + b_i, conv_f_b + b_f, conv_c_b + b_c,
                            conv_o_b + b_o])[None, :]

    hn, cn = _dense_gates(x, h, c, t1p, txp, wx, wh, w1[:HALF], w1[HALF:],
                          w2[:HALF], w2[HALF:], bias, w_c_i[None, :],
                          w_c_f[None, :], w_c_o[None, :])
    return hn, cn
